# baseline (speedup 1.0000x reference)
import functools

import jax
import jax.numpy as jnp
from jax import lax
from jax.experimental import pallas as pl
from jax.experimental.pallas import tpu as pltpu
from jax.experimental.pallas import tpu_sc as plsc

D = 64
SCALE = 8.0
NBUF = 4

_info = plsc.get_sparse_core_info()
NC, NS, L = _info.num_cores, _info.num_subcores, _info.num_lanes
NW = NC * NS


def _lookup(emb, tok4):
    sb, tbn = tok4.shape[0], tok4.shape[1]
    seq = sb * 8
    nt = seq // NBUF

    mesh = plsc.VectorSubcoreMesh(core_axis_name="c", subcore_axis_name="s")

    @functools.partial(
        pl.kernel,
        mesh=mesh,
        compiler_params=pltpu.CompilerParams(
            use_tc_tiling_on_sc=False, needs_layout_passes=False),
        out_type=jax.ShapeDtypeStruct((seq, D // 8, tbn, 8, 128), jnp.float32),
        scratch_types=(
            [pltpu.VMEM((sb, 8, 128), jnp.int32)]
            + [pltpu.VMEM((128, D), jnp.float32) for _ in range(NBUF)]
            + [pltpu.VMEM((D // 8, 8, 128), jnp.float32) for _ in range(NBUF)]
            + [pltpu.SemaphoreType.DMA for _ in range(2 * NBUF)]
        ),
    )
    def k(emb_hbm, tok_hbm, out_hbm, idx_v, *bufs_and_sems):
        a_bufs = bufs_and_sems[:NBUF]
        b_bufs = bufs_and_sems[NBUF:2 * NBUF]
        gsems = bufs_and_sems[2 * NBUF:3 * NBUF]
        ssems = bufs_and_sems[3 * NBUF:]

        wid = lax.axis_index("s") * NC + lax.axis_index("c")
        for a in range(sb):
            pltpu.sync_copy(tok_hbm.at[a, wid], idx_v.at[a])

        riota = [lax.iota(jnp.int32, 16) + c * 16 for c in range(8)]

        def fire_gather(b, s):
            pltpu.async_copy(
                emb_hbm.at[idx_v.at[s // 8, s % 8]], a_bufs[b], gsems[b])

        def wait_gather(b, s):
            pltpu.make_async_copy(
                emb_hbm.at[idx_v.at[s // 8, s % 8]], a_bufs[b],
                gsems[b]).wait()

        def fire_store(b, s):
            pltpu.async_copy(b_bufs[b], out_hbm.at[s, :, wid], ssems[b])

        def wait_store(b, s):
            pltpu.make_async_copy(
                b_bufs[b], out_hbm.at[s, :, wid], ssems[b]).wait()

        def transpose_scale(b):
            src, dst = a_bufs[b], b_bufs[b]

            def dbody(d, _):
                col = jnp.full((16,), d, jnp.int32)
                for c in range(8):
                    v = plsc.load_gather(src, [riota[c], col])
                    dst[d // 8, d % 8, pl.ds(c * 16, 16)] = v * SCALE
                return 0

            lax.fori_loop(0, D, dbody, 0)

        for b in range(NBUF):
            fire_gather(b, b)

        for b in range(NBUF):
            wait_gather(b, b)
            transpose_scale(b)
            fire_gather(b, NBUF + b)
            fire_store(b, b)

        def step(t, _):
            for b in range(NBUF):
                s = t * NBUF + b
                wait_gather(b, s)
                wait_store(b, s - NBUF)
                transpose_scale(b)
                fire_gather(b, s + NBUF)
                fire_store(b, s)
            return 0

        lax.fori_loop(1, nt - 1, step, 0)

        for b in range(NBUF):
            s = (nt - 1) * NBUF + b
            wait_gather(b, s)
            wait_store(b, s - NBUF)
            transpose_scale(b)
            fire_store(b, s)

        for b in range(NBUF):
            wait_store(b, (nt - 1) * NBUF + b)

    return k(emb, tok4)


def kernel(tokens, embedding):
    bsz, seq = tokens.shape
    sb, tbn = seq // 8, bsz // 128
    tok4 = (tokens.astype(jnp.int32).T
            .reshape(sb, 8, tbn, 128).transpose(0, 2, 1, 3))
    y5 = _lookup(embedding, tok4)
    return jnp.transpose(y5, (2, 4, 0, 1, 3)).reshape(bsz, seq, D)


# speedup vs baseline: 1.5794x; 1.5794x over previous
import functools

import jax
import jax.numpy as jnp
from jax import lax
from jax.experimental import pallas as pl
from jax.experimental.pallas import tpu as pltpu
from jax.experimental.pallas import tpu_sc as plsc

D = 64
SCALE = 8.0
NBUF = 4
RU = 4
GS0 = 104

_info = plsc.get_sparse_core_info()
NC, NS, L = _info.num_cores, _info.num_subcores, _info.num_lanes
NW = NC * NS


def _lookup(emb, tokens):
    bsz, seq = tokens.shape
    rows_per_w = bsz // NW
    gs1 = seq - GS0
    nt = rows_per_w // NBUF

    mesh = plsc.VectorSubcoreMesh(core_axis_name="c", subcore_axis_name="s")

    @functools.partial(
        pl.kernel,
        mesh=mesh,
        compiler_params=pltpu.CompilerParams(use_tc_tiling_on_sc=False),
        out_type=jax.ShapeDtypeStruct((bsz, seq, D), jnp.float32),
        scratch_types=(
            [pltpu.VMEM((rows_per_w, seq), jnp.int32)]
            + [pltpu.VMEM((seq, D), jnp.float32) for _ in range(2 * NBUF)]
            + [pltpu.SemaphoreType.DMA for _ in range(2 * NBUF)]
        ),
    )
    def k(emb_hbm, tok_hbm, out_hbm, idx_v, *bufs_and_sems):
        a_bufs = bufs_and_sems[:NBUF]
        b_bufs = bufs_and_sems[NBUF:2 * NBUF]
        gsems = bufs_and_sems[2 * NBUF:3 * NBUF]
        ssems = bufs_and_sems[3 * NBUF:]

        wid = lax.axis_index("s") * NC + lax.axis_index("c")
        r0 = wid * rows_per_w
        pltpu.sync_copy(tok_hbm.at[pl.ds(r0, rows_per_w)], idx_v)

        def fire_gather(b, r):
            pltpu.async_copy(
                emb_hbm.at[idx_v.at[r, pl.ds(0, GS0)]],
                a_bufs[b].at[pl.ds(0, GS0)], gsems[b])
            pltpu.async_copy(
                emb_hbm.at[idx_v.at[r, pl.ds(GS0, gs1)]],
                a_bufs[b].at[pl.ds(GS0, gs1)], gsems[b])

        def wait_gather(b, r):
            pltpu.make_async_copy(
                emb_hbm.at[idx_v.at[r, pl.ds(0, GS0)]],
                a_bufs[b].at[pl.ds(0, GS0)], gsems[b]).wait()
            pltpu.make_async_copy(
                emb_hbm.at[idx_v.at[r, pl.ds(GS0, gs1)]],
                a_bufs[b].at[pl.ds(GS0, gs1)], gsems[b]).wait()

        def fire_store(b, r):
            pltpu.async_copy(b_bufs[b], out_hbm.at[r0 + r], ssems[b])

        def wait_store(b, r):
            pltpu.make_async_copy(
                b_bufs[b], out_hbm.at[r0 + r], ssems[b]).wait()

        def scale(b):
            src, dst = a_bufs[b], b_bufs[b]

            def rows(i, _):
                base = i * RU
                for rr in range(RU):
                    for j in range(D // L):
                        dst[base + rr, pl.ds(j * L, L)] = (
                            src[base + rr, pl.ds(j * L, L)] * SCALE)
                return 0

            lax.fori_loop(0, seq // RU, rows, 0)

        for b in range(NBUF):
            fire_gather(b, b)

        for b in range(NBUF):
            wait_gather(b, b)
            scale(b)
            fire_gather(b, NBUF + b)
            fire_store(b, b)

        def step(t, _):
            for b in range(NBUF):
                r = t * NBUF + b
                wait_gather(b, r)
                wait_store(b, r - NBUF)
                scale(b)
                fire_gather(b, r + NBUF)
                fire_store(b, r)
            return 0

        lax.fori_loop(1, nt - 1, step, 0)

        for b in range(NBUF):
            r = (nt - 1) * NBUF + b
            wait_gather(b, r)
            wait_store(b, r - NBUF)
            scale(b)
            fire_store(b, r)

        for b in range(NBUF):
            wait_store(b, (nt - 1) * NBUF + b)

    return k(emb, tokens)


def kernel(tokens, embedding):
    return _lookup(embedding, tokens.astype(jnp.int32))
